# Initial kernel scaffold; baseline (speedup 1.0000x reference)
import jax
import jax.numpy as jnp
from jax.experimental import pallas as pl


def kernel(x, lower, _, W1_0, W1_1, b1, W2_0, W2_1, b2, We, be, Wo, bo):
    raise NotImplementedError("write your pallas kernel here")


# speedup vs baseline: 1.0294x; 1.0294x over previous
import functools

import jax
import jax.numpy as jnp
from jax.experimental import pallas as pl
from jax.experimental.pallas import tpu as pltpu

TN1 = 256
TN2 = 256


def _l1_kernel(B, TN, lower_ref, xT_ref, W10_ref, W11_ref, b1_ref, h_ref):
    i = pl.program_id(0)
    L = lower_ref[...]
    xT = xT_ref[...]
    s = jnp.dot(L, xT, preferred_element_type=jnp.float32)
    xr = xT_ref[pl.ds(i * TN, TN), :]
    W10 = W10_ref[...]
    W11 = W11_ref[...]
    b1 = b1_ref[...]
    pieces = []
    for b in range(B):
        hb = xr[:, b:b + 1] * W10 + s[:, b:b + 1] * W11 + b1
        pieces.append(jnp.maximum(hb, 0.0))
    h_ref[...] = jnp.concatenate(pieces, axis=1)


def _l2_kernel(B, HD, TN, lower_ref, h_ref, W20_ref, W21_ref, b2_ref,
               We_ref, be_ref, Wo_ref, bo_ref, out_ref, m_ref):
    i = pl.program_id(0)
    nsteps = pl.num_programs(0)
    L = lower_ref[...]
    agg = jnp.dot(L, h_ref[...], preferred_element_type=jnp.float32)
    Hi = h_ref[pl.ds(i * TN, TN), :]
    A = agg.reshape(TN * B, HD)
    Hf = Hi.reshape(TN * B, HD)
    G = (jnp.dot(Hf, W20_ref[...], preferred_element_type=jnp.float32)
         + jnp.dot(A, W21_ref[...], preferred_element_type=jnp.float32)
         + b2_ref[...])
    G = jnp.maximum(G, 0.0)
    Gm = jnp.max(G.reshape(TN, B * HD), axis=0, keepdims=True)

    @pl.when(i == 0)
    def _():
        m_ref[...] = Gm

    @pl.when(i > 0)
    def _():
        m_ref[...] = jnp.maximum(m_ref[...], Gm)

    @pl.when(i == nsteps - 1)
    def _():
        mm = m_ref[...].reshape(B, HD)
        t = jnp.dot(mm, We_ref[...], preferred_element_type=jnp.float32)
        t = jnp.maximum(t + be_ref[...], 0.0)
        out_ref[...] = (jnp.dot(t, Wo_ref[...],
                                preferred_element_type=jnp.float32)
                        + bo_ref[...])


def kernel(x, lower, _, W1_0, W1_1, b1, W2_0, W2_1, b2, We, be, Wo, bo):
    B, N, _d = x.shape
    HD = W1_0.shape[1]
    HFF = We.shape[1]
    NC = Wo.shape[1]

    xT = x[:, :, 0].T
    b1r = b1.reshape(1, HD)
    b2r = b2.reshape(1, HD)
    ber = be.reshape(1, HFF)
    bor = bo.reshape(1, NC)

    h2d = pl.pallas_call(
        functools.partial(_l1_kernel, B, TN1),
        grid=(N // TN1,),
        in_specs=[
            pl.BlockSpec((TN1, N), lambda i: (i, 0)),
            pl.BlockSpec((N, B), lambda i: (0, 0)),
            pl.BlockSpec((1, HD), lambda i: (0, 0)),
            pl.BlockSpec((1, HD), lambda i: (0, 0)),
            pl.BlockSpec((1, HD), lambda i: (0, 0)),
        ],
        out_specs=pl.BlockSpec((TN1, B * HD), lambda i: (i, 0)),
        out_shape=jax.ShapeDtypeStruct((N, B * HD), jnp.float32),
    )(lower, xT, W1_0, W1_1, b1r)

    out = pl.pallas_call(
        functools.partial(_l2_kernel, B, HD, TN2),
        grid=(N // TN2,),
        in_specs=[
            pl.BlockSpec((TN2, N), lambda i: (i, 0)),
            pl.BlockSpec((N, B * HD), lambda i: (0, 0)),
            pl.BlockSpec((HD, HD), lambda i: (0, 0)),
            pl.BlockSpec((HD, HD), lambda i: (0, 0)),
            pl.BlockSpec((1, HD), lambda i: (0, 0)),
            pl.BlockSpec((HD, HFF), lambda i: (0, 0)),
            pl.BlockSpec((1, HFF), lambda i: (0, 0)),
            pl.BlockSpec((HFF, NC), lambda i: (0, 0)),
            pl.BlockSpec((1, NC), lambda i: (0, 0)),
        ],
        out_specs=pl.BlockSpec((B, NC), lambda i: (0, 0)),
        out_shape=jax.ShapeDtypeStruct((B, NC), jnp.float32),
        scratch_shapes=[pltpu.VMEM((1, B * HD), jnp.float32)],
    )(lower, h2d, W2_0, W2_1, b2r, We, ber, Wo, bor)

    return out


# speedup vs baseline: 1.0549x; 1.0248x over previous
import functools

import jax
import jax.numpy as jnp
from jax.experimental import pallas as pl
from jax.experimental.pallas import tpu as pltpu

TN1 = 256
TN2 = 256


def _l1_kernel(B, TN, lower_ref, xT_ref, W10_ref, W11_ref, b1_ref, h_ref):
    i = pl.program_id(0)
    L = lower_ref[...]
    xT = xT_ref[...]
    s = jnp.dot(L, xT, preferred_element_type=jnp.float32)
    xr = xT_ref[pl.ds(i * TN, TN), :]
    W10 = W10_ref[...]
    W11 = W11_ref[...]
    b1 = b1_ref[...]
    pieces = []
    for b in range(B):
        hb = xr[:, b:b + 1] * W10 + s[:, b:b + 1] * W11 + b1
        pieces.append(jnp.maximum(hb, 0.0))
    h_ref[...] = jnp.concatenate(pieces, axis=1).astype(jnp.bfloat16)


def _l2_kernel(B, HD, TN, lower_ref, h_ref, W20_ref, W21_ref, b2_ref,
               We_ref, be_ref, Wo_ref, bo_ref, out_ref, m_ref):
    i = pl.program_id(0)
    nsteps = pl.num_programs(0)
    L = lower_ref[...].astype(jnp.bfloat16)
    agg = jnp.dot(L, h_ref[...], preferred_element_type=jnp.float32)
    Hi = h_ref[pl.ds(i * TN, TN), :]
    A = agg.reshape(TN * B, HD).astype(jnp.bfloat16)
    Hf = Hi.reshape(TN * B, HD)
    G = (jnp.dot(Hf, W20_ref[...], preferred_element_type=jnp.float32)
         + jnp.dot(A, W21_ref[...], preferred_element_type=jnp.float32)
         + b2_ref[...])
    G = jnp.maximum(G, 0.0)
    Gm = jnp.max(G.reshape(TN, B * HD), axis=0, keepdims=True)

    @pl.when(i == 0)
    def _():
        m_ref[...] = Gm

    @pl.when(i > 0)
    def _():
        m_ref[...] = jnp.maximum(m_ref[...], Gm)

    @pl.when(i == nsteps - 1)
    def _():
        mm = m_ref[...].reshape(B, HD)
        t = jnp.dot(mm, We_ref[...], preferred_element_type=jnp.float32)
        t = jnp.maximum(t + be_ref[...], 0.0)
        out_ref[...] = (jnp.dot(t, Wo_ref[...],
                                preferred_element_type=jnp.float32)
                        + bo_ref[...])


def kernel(x, lower, _, W1_0, W1_1, b1, W2_0, W2_1, b2, We, be, Wo, bo):
    B, N, _d = x.shape
    HD = W1_0.shape[1]
    HFF = We.shape[1]
    NC = Wo.shape[1]

    xT = x[:, :, 0].T
    b1r = b1.reshape(1, HD)
    b2r = b2.reshape(1, HD)
    ber = be.reshape(1, HFF)
    bor = bo.reshape(1, NC)

    h2d = pl.pallas_call(
        functools.partial(_l1_kernel, B, TN1),
        grid=(N // TN1,),
        in_specs=[
            pl.BlockSpec((TN1, N), lambda i: (i, 0)),
            pl.BlockSpec((N, B), lambda i: (0, 0)),
            pl.BlockSpec((1, HD), lambda i: (0, 0)),
            pl.BlockSpec((1, HD), lambda i: (0, 0)),
            pl.BlockSpec((1, HD), lambda i: (0, 0)),
        ],
        out_specs=pl.BlockSpec((TN1, B * HD), lambda i: (i, 0)),
        out_shape=jax.ShapeDtypeStruct((N, B * HD), jnp.bfloat16),
    )(lower, xT, W1_0, W1_1, b1r)

    out = pl.pallas_call(
        functools.partial(_l2_kernel, B, HD, TN2),
        grid=(N // TN2,),
        in_specs=[
            pl.BlockSpec((TN2, N), lambda i: (i, 0)),
            pl.BlockSpec((N, B * HD), lambda i: (0, 0)),
            pl.BlockSpec((HD, HD), lambda i: (0, 0)),
            pl.BlockSpec((HD, HD), lambda i: (0, 0)),
            pl.BlockSpec((1, HD), lambda i: (0, 0)),
            pl.BlockSpec((HD, HFF), lambda i: (0, 0)),
            pl.BlockSpec((1, HFF), lambda i: (0, 0)),
            pl.BlockSpec((HFF, NC), lambda i: (0, 0)),
            pl.BlockSpec((1, NC), lambda i: (0, 0)),
        ],
        out_specs=pl.BlockSpec((B, NC), lambda i: (0, 0)),
        out_shape=jax.ShapeDtypeStruct((B, NC), jnp.float32),
        scratch_shapes=[pltpu.VMEM((1, B * HD), jnp.float32)],
    )(lower, h2d, W2_0.astype(jnp.bfloat16), W2_1.astype(jnp.bfloat16),
      b2r, We, ber, Wo, bor)

    return out
